# baseline (speedup 1.0000x reference)
import functools

import jax
import jax.numpy as jnp
from jax import lax
from jax.experimental import pallas as pl
from jax.experimental.pallas import tpu as pltpu
from jax.experimental.pallas import tpu_sc as plsc

L = 16


def _make_kernel(C, H, W, K0, OH, K1, OW):
    NW = 32
    RPW = C * OH // NW
    BO = 8
    NB = RPW // BO

    mesh = plsc.VectorSubcoreMesh(core_axis_name="c", subcore_axis_name="s")

    @functools.partial(
        pl.kernel,
        out_type=jax.ShapeDtypeStruct((C * OH, OW), jnp.float32),
        mesh=mesh,
        compiler_params=pltpu.CompilerParams(
            needs_layout_passes=False, use_tc_tiling_on_sc=False),
        scratch_types=[
            pltpu.VMEM((RPW * K0,), jnp.int32),
            pltpu.VMEM((OH, K0, L), jnp.float32),
            pltpu.VMEM((K1, OW), jnp.int32),
            pltpu.VMEM((K1, OW), jnp.float32),
            pltpu.VMEM((BO * K0, W), jnp.float32),
            pltpu.VMEM((W,), jnp.float32),
            pltpu.VMEM((BO, OW), jnp.float32),
            pltpu.SemaphoreType.DMA,
        ],
    )
    def k(x_hbm, gidx_hbm, w0b_hbm, fov1_hbm, w1_hbm, out_hbm,
          gidx_v, w0b_v, fov1_v, w1_v, rows_v, vacc_v, outbuf_v, sem):
        wid = lax.axis_index("s") * 2 + lax.axis_index("c")
        pltpu.sync_copy(gidx_hbm.at[wid], gidx_v)
        pltpu.sync_copy(w0b_hbm, w0b_v)
        pltpu.sync_copy(fov1_hbm, fov1_v)
        pltpu.sync_copy(w1_hbm, w1_v)

        def block_body(g, _):
            pltpu.async_copy(
                x_hbm.at[gidx_v.at[pl.ds(g * (BO * K0), BO * K0)]],
                rows_v, sem).wait()

            def one_row(r, carry):
                i = lax.rem(g * BO + r, OH)
                wv = [w0b_v[i, t, :] for t in range(K0)]
                for ch in range(W // L):
                    sl = pl.ds(ch * L, L)
                    acc = rows_v[r * K0, sl] * wv[0]
                    for t in range(1, K0):
                        acc = acc + rows_v[r * K0 + t, sl] * wv[t]
                    vacc_v[sl] = acc
                for jg in range(OW // L):
                    slj = pl.ds(jg * L, L)
                    vo = plsc.load_gather(vacc_v, [fov1_v[0, slj]]) * w1_v[0, slj]
                    for t in range(1, K1):
                        vo = vo + (plsc.load_gather(vacc_v, [fov1_v[t, slj]])
                                   * w1_v[t, slj])
                    outbuf_v[r, slj] = vo
                return carry

            lax.fori_loop(0, BO, one_row, None)
            pltpu.sync_copy(outbuf_v,
                            out_hbm.at[pl.ds(wid * RPW + g * BO, BO)])
            return _

        lax.fori_loop(0, NB, block_body, None)

    return k


def kernel(in_tensor, fov0, w0, fov1, w1):
    C, H, W = in_tensor.shape
    K0, OH = fov0.shape
    K1, OW = fov1.shape
    NW = 32

    x2d = in_tensor.reshape(C * H, W)
    fov0i = fov0.astype(jnp.int32)
    fov1i = fov1.astype(jnp.int32)
    w0m = w0.reshape(K0, OH).astype(jnp.float32)
    w1m = w1.reshape(K1, OW).astype(jnp.float32)

    gidx = (jnp.arange(C, dtype=jnp.int32)[:, None, None] * H
            + fov0i.T[None, :, :])
    gidx = gidx.reshape(NW, (C * OH // NW) * K0)

    w0b = jnp.broadcast_to(w0m.T[:, :, None], (OH, K0, L))

    k = _make_kernel(C, H, W, K0, OH, K1, OW)
    out2d = k(x2d, gidx, w0b, fov1i, w1m)
    return out2d.reshape(C, OH, OW)


# speedup vs baseline: 1.2677x; 1.2677x over previous
import functools

import jax
import jax.numpy as jnp
from jax import lax
from jax.experimental import pallas as pl
from jax.experimental.pallas import tpu as pltpu
from jax.experimental.pallas import tpu_sc as plsc

L = 16


def _make_kernel(C, H, W, K0, OH, K1, OW):
    NW = 32
    RPW = C * OH // NW
    BO = 8
    NB = RPW // BO

    mesh = plsc.VectorSubcoreMesh(core_axis_name="c", subcore_axis_name="s")

    @functools.partial(
        pl.kernel,
        out_type=jax.ShapeDtypeStruct((C * OH, OW), jnp.float32),
        mesh=mesh,
        compiler_params=pltpu.CompilerParams(
            needs_layout_passes=False, use_tc_tiling_on_sc=False),
        scratch_types=[
            pltpu.VMEM((RPW * K0,), jnp.int32),
            pltpu.VMEM((OH, K0, L), jnp.float32),
            pltpu.VMEM((K1, OW), jnp.int32),
            pltpu.VMEM((K1, OW), jnp.float32),
            pltpu.VMEM((2, BO * K0, W), jnp.float32),
            pltpu.VMEM((BO * W,), jnp.float32),
            pltpu.VMEM((BO, OW), jnp.float32),
            pltpu.SemaphoreType.DMA,
            pltpu.SemaphoreType.DMA,
        ],
    )
    def k(x_hbm, gidx_hbm, w0b_hbm, fov1_hbm, w1_hbm, out_hbm,
          gidx_v, w0b_v, fov1_v, w1_v, rows_v, vacc_v, outbuf_v, sem0, sem1):
        wid = lax.axis_index("s") * 2 + lax.axis_index("c")
        pltpu.sync_copy(gidx_hbm.at[wid], gidx_v)
        pltpu.sync_copy(w0b_hbm, w0b_v)
        pltpu.sync_copy(fov1_hbm, fov1_v)
        pltpu.sync_copy(w1_hbm, w1_v)

        def gsrc(g):
            return x_hbm.at[gidx_v.at[pl.ds(g * (BO * K0), BO * K0)]]

        def compute_block(g, rv):
            def vrow(r, carry):
                i = lax.rem(g * BO + r, OH)
                wv = [w0b_v[i, t, :] for t in range(K0)]
                for ch in range(W // L):
                    sl = pl.ds(ch * L, L)
                    acc = rv[r * K0, sl] * wv[0]
                    for t in range(1, K0):
                        acc = acc + rv[r * K0 + t, sl] * wv[t]
                    vacc_v[pl.ds(r * W + ch * L, L)] = acc
                return carry

            lax.fori_loop(0, BO, vrow, None)

            for jg in range(OW // L):
                slj = pl.ds(jg * L, L)
                idxs = [fov1_v[t, slj] for t in range(K1)]
                ws = [w1_v[t, slj] for t in range(K1)]

                def hrow(r, carry):
                    rw = r * W
                    vo = plsc.load_gather(vacc_v, [idxs[0] + rw]) * ws[0]
                    for t in range(1, K1):
                        vo = vo + plsc.load_gather(vacc_v, [idxs[t] + rw]) * ws[t]
                    outbuf_v[r, slj] = vo
                    return carry

                lax.fori_loop(0, BO, hrow, None)

            pltpu.sync_copy(outbuf_v,
                            out_hbm.at[pl.ds(wid * RPW + g * BO, BO)])

        pltpu.async_copy(gsrc(0), rows_v.at[0], sem0)

        def pair(p, carry):
            g0 = 2 * p
            pltpu.async_copy(gsrc(g0 + 1), rows_v.at[1], sem1)
            pltpu.make_async_copy(gsrc(g0), rows_v.at[0], sem0).wait()
            compute_block(g0, rows_v.at[0])

            @pl.when(p < NB // 2 - 1)
            def _():
                pltpu.async_copy(gsrc(g0 + 2), rows_v.at[0], sem0)

            pltpu.make_async_copy(gsrc(g0 + 1), rows_v.at[1], sem1).wait()
            compute_block(g0 + 1, rows_v.at[1])
            return carry

        lax.fori_loop(0, NB // 2, pair, None)

    return k


def kernel(in_tensor, fov0, w0, fov1, w1):
    C, H, W = in_tensor.shape
    K0, OH = fov0.shape
    K1, OW = fov1.shape
    NW = 32

    x2d = in_tensor.reshape(C * H, W)
    fov0i = fov0.astype(jnp.int32)
    fov1i = fov1.astype(jnp.int32)
    w0m = w0.reshape(K0, OH).astype(jnp.float32)
    w1m = w1.reshape(K1, OW).astype(jnp.float32)

    gidx = (jnp.arange(C, dtype=jnp.int32)[:, None, None] * H
            + fov0i.T[None, :, :])
    gidx = gidx.reshape(NW, (C * OH // NW) * K0)

    w0b = jnp.broadcast_to(w0m.T[:, :, None], (OH, K0, L))

    k = _make_kernel(C, H, W, K0, OH, K1, OW)
    out2d = k(x2d, gidx, w0b, fov1i, w1m)
    return out2d.reshape(C, OH, OW)


# speedup vs baseline: 1.3802x; 1.0888x over previous
import functools

import jax
import jax.numpy as jnp
from jax import lax
from jax.experimental import pallas as pl
from jax.experimental.pallas import tpu as pltpu
from jax.experimental.pallas import tpu_sc as plsc

L = 16


def _make_kernel(C, H, W, K0, OH, K1, OW):
    NW = 32
    RPW = C * OH // NW
    BO = 8
    NB = RPW // BO

    mesh = plsc.VectorSubcoreMesh(core_axis_name="c", subcore_axis_name="s")

    @functools.partial(
        pl.kernel,
        out_type=jax.ShapeDtypeStruct((C * OH, OW), jnp.float32),
        mesh=mesh,
        compiler_params=pltpu.CompilerParams(
            needs_layout_passes=False, use_tc_tiling_on_sc=False),
        scratch_types=[
            pltpu.VMEM((RPW * K0,), jnp.int32),
            pltpu.VMEM((OH, K0, L), jnp.float32),
            pltpu.VMEM((K1, OW), jnp.int32),
            pltpu.VMEM((K1, OW), jnp.float32),
            pltpu.VMEM((2, BO * K0, W), jnp.float32),
            pltpu.VMEM((BO * W,), jnp.float32),
            pltpu.VMEM((2, BO, OW), jnp.float32),
            pltpu.SemaphoreType.DMA,
            pltpu.SemaphoreType.DMA,
            pltpu.SemaphoreType.DMA,
            pltpu.SemaphoreType.DMA,
        ],
    )
    def k(x_hbm, gidx_hbm, w0b_hbm, fov1_hbm, w1_hbm, out_hbm,
          gidx_v, w0b_v, fov1_v, w1_v, rows_v, vacc_v, outbuf_v,
          sem0, sem1, osem0, osem1):
        wid = lax.axis_index("s") * 2 + lax.axis_index("c")
        pltpu.sync_copy(gidx_hbm.at[wid], gidx_v)
        pltpu.sync_copy(w0b_hbm, w0b_v)
        pltpu.sync_copy(fov1_hbm, fov1_v)
        pltpu.sync_copy(w1_hbm, w1_v)

        def gsrc(g):
            return x_hbm.at[gidx_v.at[pl.ds(g * (BO * K0), BO * K0)]]

        def odst(g):
            return out_hbm.at[pl.ds(wid * RPW + g * BO, BO)]

        def compute_block(g, rv, ob, osem, p):
            def vrow(r, carry):
                i = lax.rem(g * BO + r, OH)
                wv = [w0b_v[i, t, :] for t in range(K0)]
                for ch in range(W // L):
                    sl = pl.ds(ch * L, L)
                    acc = rv[r * K0, sl] * wv[0]
                    for t in range(1, K0):
                        acc = acc + rv[r * K0 + t, sl] * wv[t]
                    vacc_v[pl.ds(r * W + ch * L, L)] = acc
                return carry

            lax.fori_loop(0, BO, vrow, None)

            @pl.when(p > 0)
            def _():
                pltpu.make_async_copy(ob, odst(g - 2), osem).wait()

            def hgrp(jg, carry):
                slj = pl.ds(jg * L, L)
                idxs = [fov1_v[t, slj] for t in range(K1)]
                ws = [w1_v[t, slj] for t in range(K1)]
                for r in range(BO):
                    rw = r * W
                    vo = plsc.load_gather(vacc_v, [idxs[0] + rw]) * ws[0]
                    for t in range(1, K1):
                        vo = vo + plsc.load_gather(vacc_v, [idxs[t] + rw]) * ws[t]
                    ob[r, slj] = vo
                return carry

            lax.fori_loop(0, OW // L, hgrp, None)
            pltpu.async_copy(ob, odst(g), osem)

        pltpu.async_copy(gsrc(0), rows_v.at[0], sem0)

        def pair(p, carry):
            g0 = 2 * p
            pltpu.async_copy(gsrc(g0 + 1), rows_v.at[1], sem1)
            pltpu.make_async_copy(gsrc(g0), rows_v.at[0], sem0).wait()
            compute_block(g0, rows_v.at[0], outbuf_v.at[0], osem0, p)

            @pl.when(p < NB // 2 - 1)
            def _():
                pltpu.async_copy(gsrc(g0 + 2), rows_v.at[0], sem0)

            pltpu.make_async_copy(gsrc(g0 + 1), rows_v.at[1], sem1).wait()
            compute_block(g0 + 1, rows_v.at[1], outbuf_v.at[1], osem1, p)
            return carry

        lax.fori_loop(0, NB // 2, pair, None)
        pltpu.make_async_copy(outbuf_v.at[0], odst(NB - 2), osem0).wait()
        pltpu.make_async_copy(outbuf_v.at[1], odst(NB - 1), osem1).wait()

    return k


def kernel(in_tensor, fov0, w0, fov1, w1):
    C, H, W = in_tensor.shape
    K0, OH = fov0.shape
    K1, OW = fov1.shape
    NW = 32

    x2d = in_tensor.reshape(C * H, W)
    fov0i = fov0.astype(jnp.int32)
    fov1i = fov1.astype(jnp.int32)
    w0m = w0.reshape(K0, OH).astype(jnp.float32)
    w1m = w1.reshape(K1, OW).astype(jnp.float32)

    gidx = (jnp.arange(C, dtype=jnp.int32)[:, None, None] * H
            + fov0i.T[None, :, :])
    gidx = gidx.reshape(NW, (C * OH // NW) * K0)

    w0b = jnp.broadcast_to(w0m.T[:, :, None], (OH, K0, L))

    k = _make_kernel(C, H, W, K0, OH, K1, OW)
    out2d = k(x2d, gidx, w0b, fov1i, w1m)
    return out2d.reshape(C, OH, OW)


# speedup vs baseline: 1.4459x; 1.0476x over previous
import functools

import jax
import jax.numpy as jnp
from jax import lax
from jax.experimental import pallas as pl
from jax.experimental.pallas import tpu as pltpu
from jax.experimental.pallas import tpu_sc as plsc

L = 16
L2 = 32


def _make_kernel(C, H, W, K0, OH, K1, OW):
    NW = 32
    RPW = C * OH // NW
    BO = 8
    NB = RPW // BO

    mesh = plsc.VectorSubcoreMesh(core_axis_name="c", subcore_axis_name="s")

    @functools.partial(
        pl.kernel,
        out_type=jax.ShapeDtypeStruct((C * OH, OW), jnp.float32),
        mesh=mesh,
        compiler_params=pltpu.CompilerParams(
            needs_layout_passes=False, use_tc_tiling_on_sc=False),
        scratch_types=[
            pltpu.VMEM((RPW * K0,), jnp.int32),
            pltpu.VMEM((OH, K0, L2), jnp.bfloat16),
            pltpu.VMEM((K1, OW), jnp.int32),
            pltpu.VMEM((K1, OW), jnp.float32),
            pltpu.VMEM((2, BO * K0, W), jnp.bfloat16),
            pltpu.VMEM((BO * W,), jnp.float32),
            pltpu.VMEM((2, BO, OW), jnp.float32),
            pltpu.SemaphoreType.DMA,
            pltpu.SemaphoreType.DMA,
            pltpu.SemaphoreType.DMA,
            pltpu.SemaphoreType.DMA,
        ],
    )
    def k(x_hbm, gidx_hbm, w0b_hbm, fov1_hbm, w1_hbm, out_hbm,
          gidx_v, w0b_v, fov1_v, w1_v, rows_v, vacc_v, outbuf_v,
          sem0, sem1, osem0, osem1):
        wid = lax.axis_index("s") * 2 + lax.axis_index("c")
        pltpu.sync_copy(gidx_hbm.at[wid], gidx_v)
        pltpu.sync_copy(w0b_hbm, w0b_v)
        pltpu.sync_copy(fov1_hbm, fov1_v)
        pltpu.sync_copy(w1_hbm, w1_v)

        def gsrc(g):
            return x_hbm.at[gidx_v.at[pl.ds(g * (BO * K0), BO * K0)]]

        def odst(g):
            return out_hbm.at[pl.ds(wid * RPW + g * BO, BO)]

        def compute_block(g, rv, ob, osem, p):
            def vrow(r, carry):
                i = lax.rem(g * BO + r, OH)
                wv = [w0b_v[i, t, :] for t in range(K0)]

                def accum(sl):
                    m = [rv[r * K0 + t, sl] * wv[t] for t in range(K0)]
                    while len(m) > 1:
                        m = [a + b for a, b in zip(m[::2], m[1::2])]
                    return m[0]

                for cp in range(W // L2):
                    acc = accum(pl.ds(cp * L2, L2))
                    ai = plsc.bitcast(acc, jnp.int32)
                    lo = plsc.bitcast(ai << 16, jnp.float32)
                    hi = plsc.bitcast(ai & jnp.int32(-65536), jnp.float32)
                    vacc_v[pl.ds(r * W + cp * L2, L)] = lo
                    vacc_v[pl.ds(r * W + cp * L2 + L, L)] = hi
                return carry

            lax.fori_loop(0, BO, vrow, None)

            @pl.when(p > 0)
            def _():
                pltpu.make_async_copy(ob, odst(g - 2), osem).wait()

            def hgrp(jg, carry):
                slj = pl.ds(jg * L, L)
                idxs = [fov1_v[t, slj] for t in range(K1)]
                ws = [w1_v[t, slj] for t in range(K1)]
                for r in range(BO):
                    rw = r * W
                    vo = plsc.load_gather(vacc_v, [idxs[0] + rw]) * ws[0]
                    for t in range(1, K1):
                        vo = vo + plsc.load_gather(vacc_v, [idxs[t] + rw]) * ws[t]
                    ob[r, slj] = vo
                return carry

            lax.fori_loop(0, OW // L, hgrp, None)
            pltpu.async_copy(ob, odst(g), osem)

        pltpu.async_copy(gsrc(0), rows_v.at[0], sem0)

        def pair(p, carry):
            g0 = 2 * p
            pltpu.async_copy(gsrc(g0 + 1), rows_v.at[1], sem1)
            pltpu.make_async_copy(gsrc(g0), rows_v.at[0], sem0).wait()
            compute_block(g0, rows_v.at[0], outbuf_v.at[0], osem0, p)

            @pl.when(p < NB // 2 - 1)
            def _():
                pltpu.async_copy(gsrc(g0 + 2), rows_v.at[0], sem0)

            pltpu.make_async_copy(gsrc(g0 + 1), rows_v.at[1], sem1).wait()
            compute_block(g0 + 1, rows_v.at[1], outbuf_v.at[1], osem1, p)
            return carry

        lax.fori_loop(0, NB // 2, pair, None)
        pltpu.make_async_copy(outbuf_v.at[0], odst(NB - 2), osem0).wait()
        pltpu.make_async_copy(outbuf_v.at[1], odst(NB - 1), osem1).wait()

    return k


def kernel(in_tensor, fov0, w0, fov1, w1):
    C, H, W = in_tensor.shape
    K0, OH = fov0.shape
    K1, OW = fov1.shape
    NW = 32

    x2d = in_tensor.reshape(C * H, W).astype(jnp.bfloat16)
    fov0i = fov0.astype(jnp.int32)
    fov1i = fov1.astype(jnp.int32)
    w0m = w0.reshape(K0, OH).astype(jnp.float32)
    w1m = w1.reshape(K1, OW).astype(jnp.float32)

    gidx = (jnp.arange(C, dtype=jnp.int32)[:, None, None] * H
            + fov0i.T[None, :, :])
    gidx = gidx.reshape(NW, (C * OH // NW) * K0)

    w0b = jnp.broadcast_to(
        w0m.T.astype(jnp.bfloat16)[:, :, None], (OH, K0, L2))

    fov1q = ((fov1i // L2) * L2 + (fov1i % 2) * L + (fov1i % L2) // 2)

    k = _make_kernel(C, H, W, K0, OH, K1, OW)
    out2d = k(x2d, gidx, w0b, fov1q, w1m)
    return out2d.reshape(C, OH, OW)
